# baseline (speedup 1.0000x reference)
import functools

import jax
import jax.numpy as jnp
from jax import lax
from jax.experimental import pallas as pl
from jax.experimental.pallas import tpu as pltpu
from jax.experimental.pallas import tpu_sc as plsc

KNB = 16
Q = 256
D = 128
NKEYS = 100000
BK = 5000
NSTEPS = NKEYS // BK
NFULL = BK // 128
NTAIL = BK - NFULL * 128
NCH = NFULL + 1
NSLOT = 2 * NSTEPS
PADVAL = 1e30
BIGI = 2**30

NC = 2
NS = 16
NW = NC * NS
BPW = (Q * KNB) // NW


def _merge(a, b):
    take = b[0] < a[0]
    return jnp.where(take, b[0], a[0]), jnp.where(take, b[1], a[1])


def _tree_fold(pairs, fn):
    while len(pairs) > 1:
        nxt = [fn(pairs[j], pairs[j + 1]) for j in range(0, len(pairs) - 1, 2)]
        if len(pairs) % 2:
            nxt.append(pairs[-1])
        pairs = nxt
    return pairs[0]


def _fast_body(q2_ref, x_ref, k2_ref, ks_ref, outd_ref, outi_ref, violq_ref,
               sqb, vb, ib, m3b, t5v, t5i, resv, resi):
    i = pl.program_id(0)

    @pl.when(i < NSTEPS)
    def _compute():
        x = x_ref[...]
        ks = ks_ref[...]
        k2 = k2_ref[0]
        q2 = q2_ref[...]
        dot2 = lax.dot_general(x, ks, (((1,), (1,)), ((), ())),
                               preferred_element_type=jnp.float32)
        sqb[i % 2] = (q2 + k2) + dot2

    @pl.when(i > 0)
    def _fold():
        b = i - 1
        sq = sqb[(i - 1) % 2]

        chunks = [sq[:, 128 * c:128 * (c + 1)] for c in range(NFULL)]
        chunks.append(jnp.concatenate(
            [sq[:, NFULL * 128:BK],
             jnp.full((Q, 128 - NTAIL), PADVAL, jnp.float32)], axis=1))
        m1, r1 = _tree_fold([(chunks[c], c) for c in range(NCH)], _merge)
        masked = [jnp.where(r1 == c, jnp.inf, chunks[c]) for c in range(NCH)]
        m2, r2 = _tree_fold([(masked[c], c) for c in range(NCH)], _merge)
        masked2 = [jnp.where(r2 == c, jnp.inf, masked[c]) for c in range(NCH)]
        m3 = _tree_fold([(masked2[c], c) for c in range(NCH)], _merge)[0]

        lane = lax.broadcasted_iota(jnp.int32, (Q, 128), 1)
        base = b * BK + lane
        vb[2 * b] = m1
        ib[2 * b] = base + r1 * 128
        vb[2 * b + 1] = m2
        ib[2 * b + 1] = base + r2 * 128
        m3b[b] = m3

    @pl.when(i == NSTEPS)
    def _final():
        inf2 = jnp.full((Q, 128), jnp.inf, jnp.float32)
        for s in range(5):
            t5v[s] = inf2
            t5i[s] = jnp.full((Q, 128), BIGI, jnp.int32)

        def _insert(j, _):
            v = vb[j]
            ix = ib[j]
            for s in range(5):
                sv = t5v[s]
                si = t5i[s]
                take = v < sv
                t5v[s] = jnp.where(take, v, sv)
                t5i[s] = jnp.where(take, ix, si)
                v = jnp.where(take, sv, v)
                ix = jnp.where(take, si, ix)
            return 0

        for j in range(NSLOT):
            _insert(j, 0)

        for s in range(5):
            t5v[s] = jnp.sqrt(jnp.maximum(t5v[s], 1e-12))

        def _extract(t, _):
            vs = [t5v[s] for s in range(4)]
            ws = [t5i[s] for s in range(4)]
            m = jnp.min(jnp.minimum(jnp.minimum(vs[0], vs[1]),
                                    jnp.minimum(vs[2], vs[3])), axis=1)
            mc = m[:, None]
            gi = jnp.minimum(
                jnp.minimum(jnp.where(vs[0] == mc, ws[0], BIGI),
                            jnp.where(vs[1] == mc, ws[1], BIGI)),
                jnp.minimum(jnp.where(vs[2] == mc, ws[2], BIGI),
                            jnp.where(vs[3] == mc, ws[3], BIGI)))
            gim = jnp.min(gi, axis=1)[:, None]
            for s in range(4):
                t5v[s] = jnp.where((vs[s] == mc) & (ws[s] == gim), jnp.inf,
                                   vs[s])
            resv[t] = mc
            resi[t] = gim
            return 0

        lax.fori_loop(0, KNB, _extract, 0)

        outd_ref[...] = jnp.concatenate([resv[t] for t in range(KNB)], axis=1)
        outi_ref[...] = jnp.concatenate([resi[t] for t in range(KNB)], axis=1)
        kth = resv[KNB - 1]
        vio = t5v[4] <= kth
        for c in range(NSTEPS):
            vio = vio | (jnp.sqrt(jnp.maximum(m3b[c], 1e-12)) <= kth)
        violq_ref[0, 0] = jnp.max(vio.astype(jnp.int32))


def _fast_call(q2, x, k2r, keys_p):
    last = NSTEPS - 1
    return pl.pallas_call(
        _fast_body,
        grid=(NSTEPS + 1,),
        in_specs=[
            pl.BlockSpec((Q, 1), lambda i: (0, 0)),
            pl.BlockSpec((Q, D), lambda i: (0, 0)),
            pl.BlockSpec((1, 1, BK), lambda i: (jnp.minimum(i, last), 0, 0)),
            pl.BlockSpec((BK, D), lambda i: (jnp.minimum(i, last), 0)),
        ],
        out_specs=[
            pl.BlockSpec((Q, KNB), lambda i: (0, 0)),
            pl.BlockSpec((Q, KNB), lambda i: (0, 0)),
            pl.BlockSpec(memory_space=pltpu.SMEM),
        ],
        out_shape=[
            jax.ShapeDtypeStruct((Q, KNB), jnp.float32),
            jax.ShapeDtypeStruct((Q, KNB), jnp.int32),
            jax.ShapeDtypeStruct((1, 1), jnp.int32),
        ],
        scratch_shapes=[
            pltpu.VMEM((2, Q, BK), jnp.float32),
            pltpu.VMEM((NSLOT, Q, 128), jnp.float32),
            pltpu.VMEM((NSLOT, Q, 128), jnp.int32),
            pltpu.VMEM((NSTEPS, Q, 128), jnp.float32),
            pltpu.VMEM((5, Q, 128), jnp.float32),
            pltpu.VMEM((5, Q, 128), jnp.int32),
            pltpu.VMEM((KNB, Q, 1), jnp.float32),
            pltpu.VMEM((KNB, Q, 1), jnp.int32),
        ],
    )(q2, x, k2r, keys_p)


def _exact_body(q2_ref, x_ref, k2_ref, ks_ref, outd_ref, outi_ref, bv, bi):
    i = pl.program_id(0)

    @pl.when(i == 0)
    def _init():
        bv[...] = jnp.full((Q, KNB), jnp.inf, jnp.float32)
        bi[...] = jnp.zeros((Q, KNB), jnp.int32)

    x = x_ref[...]
    ks = ks_ref[...]
    k2 = k2_ref[0]
    q2 = q2_ref[...]
    dot2 = lax.dot_general(x, ks, (((1,), (1,)), ((), ())),
                           preferred_element_type=jnp.float32)
    sq = (q2 + k2) + dot2
    d = jnp.sqrt(jnp.maximum(sq, 1e-12))

    W = KNB + BK
    cv = jnp.concatenate([bv[...], d], axis=1)
    iota_b = lax.broadcasted_iota(jnp.int32, (Q, BK), 1)
    ci = jnp.concatenate([bi[...], iota_b + i * BK], axis=1)
    iota_w = lax.broadcasted_iota(jnp.int32, (Q, W), 1)

    vals, idxs = [], []
    for _ in range(KNB):
        m = jnp.min(cv, axis=1)
        pos = jnp.min(jnp.where(cv == m[:, None], iota_w, W), axis=1)
        pm = iota_w == pos[:, None]
        gi = jnp.max(jnp.where(pm, ci, 0), axis=1)
        vals.append(m[:, None])
        idxs.append(gi[:, None])
        cv = jnp.where(pm, jnp.inf, cv)
    bv[...] = jnp.concatenate(vals, axis=1)
    bi[...] = jnp.concatenate(idxs, axis=1)

    @pl.when(i == NSTEPS - 1)
    def _fin():
        outd_ref[...] = bv[...]
        outi_ref[...] = bi[...]


def _exact_call(q2, x, k2r, keys_p):
    return pl.pallas_call(
        _exact_body,
        grid=(NSTEPS,),
        in_specs=[
            pl.BlockSpec((Q, 1), lambda i: (0, 0)),
            pl.BlockSpec((Q, D), lambda i: (0, 0)),
            pl.BlockSpec((1, 1, BK), lambda i: (i, 0, 0)),
            pl.BlockSpec((BK, D), lambda i: (i, 0)),
        ],
        out_specs=[
            pl.BlockSpec((Q, KNB), lambda i: (0, 0)),
            pl.BlockSpec((Q, KNB), lambda i: (0, 0)),
        ],
        out_shape=[
            jax.ShapeDtypeStruct((Q, KNB), jnp.float32),
            jax.ShapeDtypeStruct((Q, KNB), jnp.int32),
        ],
        scratch_shapes=[
            pltpu.VMEM((Q, KNB), jnp.float32),
            pltpu.VMEM((Q, KNB), jnp.int32),
        ],
    )(q2, x, k2r, keys_p)


def _gather_body(keys_hbm, cls_hbm, idx_hbm, emb_out, cls_out,
                 idx_v, rows_v, cls_v, sem_e, sem_c):
    wid = lax.axis_index("s") * NC + lax.axis_index("c")
    base = wid * BPW
    pltpu.sync_copy(idx_hbm.at[pl.ds(base, BPW)], idx_v)
    pltpu.async_copy(keys_hbm.at[idx_v], rows_v, sem_e).wait()
    pltpu.async_copy(cls_hbm.at[idx_v], cls_v, sem_c).wait()
    pltpu.sync_copy(rows_v, emb_out.at[pl.ds(base, BPW)])
    pltpu.sync_copy(cls_v, cls_out.at[pl.ds(base, BPW)])


@functools.lru_cache(maxsize=1)
def _sc_gather():
    return pl.kernel(
        _gather_body,
        out_type=[
            jax.ShapeDtypeStruct((Q * KNB, D), jnp.float32),
            jax.ShapeDtypeStruct((Q * KNB,), jnp.int32),
        ],
        mesh=plsc.VectorSubcoreMesh(core_axis_name="c", subcore_axis_name="s",
                                    num_cores=NC, num_subcores=NS),
        scratch_types=[
            pltpu.VMEM((BPW,), jnp.int32),
            pltpu.VMEM((BPW, D), jnp.float32),
            pltpu.VMEM((BPW,), jnp.int32),
            pltpu.SemaphoreType.DMA,
            pltpu.SemaphoreType.DMA,
        ],
    )


def kernel(x, keys, key_classes):
    q2 = jnp.sum(x * x, axis=1)
    k2 = jnp.sum(keys * keys, axis=1)
    q2c = q2[:, None]
    k2r = k2.reshape(NSTEPS, 1, BK)
    xm2 = -2.0 * x
    outd, outi, violq = _fast_call(q2c, xm2, k2r, keys)
    outd, outi = lax.cond(
        violq[0, 0] != 0,
        lambda: _exact_call(q2c, xm2, k2r, keys),
        lambda: (outd, outi),
    )
    emb, cls = _sc_gather()(keys, key_classes, outi.reshape(-1))
    return outd, emb.reshape(Q, KNB, D), cls.reshape(Q, KNB)


# speedup vs baseline: 1.0061x; 1.0061x over previous
import functools

import jax
import jax.numpy as jnp
from jax import lax
from jax.experimental import pallas as pl
from jax.experimental.pallas import tpu as pltpu
from jax.experimental.pallas import tpu_sc as plsc

KNB = 16
Q = 256
D = 128
NKEYS = 100000
BK = 5000
NSTEPS = NKEYS // BK
NFULL = BK // 128
NTAIL = BK - NFULL * 128
NCH = NFULL + 1
NSLOT = 2 * NSTEPS
PADVAL = 1e30
BIGI = 2**30

NC = 2
NS = 16
NW = NC * NS
BPW = (Q * KNB) // NW


def _merge(a, b):
    take = b[0] < a[0]
    return jnp.where(take, b[0], a[0]), jnp.where(take, b[1], a[1])


def _tree_fold(pairs, fn):
    while len(pairs) > 1:
        nxt = [fn(pairs[j], pairs[j + 1]) for j in range(0, len(pairs) - 1, 2)]
        if len(pairs) % 2:
            nxt.append(pairs[-1])
        pairs = nxt
    return pairs[0]


def _fast_body(q2_ref, x_ref, k2_ref, ks_ref, outd_ref, outi_ref, violq_ref,
               sqb, vb, ib, m3b, t5v, t5i, resv, resi):
    i = pl.program_id(0)

    @pl.when(i < NSTEPS)
    def _compute():
        x = x_ref[...]
        ks = ks_ref[...]
        k2 = k2_ref[0]
        q2 = q2_ref[...]
        dot2 = lax.dot_general(x, ks, (((1,), (1,)), ((), ())),
                               preferred_element_type=jnp.float32)
        sqb[i % 2] = (q2 + k2) + dot2

    @pl.when(i > 0)
    def _fold():
        b = i - 1
        sq = sqb[(i - 1) % 2]

        chunks = [sq[:, 128 * c:128 * (c + 1)] for c in range(NFULL)]
        chunks.append(jnp.concatenate(
            [sq[:, NFULL * 128:BK],
             jnp.full((Q, 128 - NTAIL), PADVAL, jnp.float32)], axis=1))
        m1, r1 = _tree_fold([(chunks[c], c) for c in range(NCH)], _merge)
        masked = [jnp.where(r1 == c, jnp.inf, chunks[c]) for c in range(NCH)]
        m2, r2 = _tree_fold([(masked[c], c) for c in range(NCH)], _merge)
        masked2 = [jnp.where(r2 == c, jnp.inf, masked[c]) for c in range(NCH)]
        m3 = _tree_fold([(masked2[c], c) for c in range(NCH)], _merge)[0]

        lane = lax.broadcasted_iota(jnp.int32, (Q, 128), 1)
        base = b * BK + lane
        vb[2 * b] = m1
        ib[2 * b] = base + r1 * 128
        vb[2 * b + 1] = m2
        ib[2 * b + 1] = base + r2 * 128
        m3b[b] = m3

    @pl.when(i == NSTEPS)
    def _final():
        inf2 = jnp.full((Q, 128), jnp.inf, jnp.float32)
        for s in range(5):
            t5v[s] = inf2
            t5i[s] = jnp.full((Q, 128), BIGI, jnp.int32)

        def _insert(j, _):
            v = vb[j]
            ix = ib[j]
            for s in range(5):
                sv = t5v[s]
                si = t5i[s]
                take = v < sv
                t5v[s] = jnp.where(take, v, sv)
                t5i[s] = jnp.where(take, ix, si)
                v = jnp.where(take, sv, v)
                ix = jnp.where(take, si, ix)
            return 0

        lax.fori_loop(0, NSLOT, _insert, 0)

        for s in range(5):
            t5v[s] = jnp.sqrt(jnp.maximum(t5v[s], 1e-12))

        def _extract(t, _):
            vs = [t5v[s] for s in range(4)]
            ws = [t5i[s] for s in range(4)]
            m = jnp.min(jnp.minimum(jnp.minimum(vs[0], vs[1]),
                                    jnp.minimum(vs[2], vs[3])), axis=1)
            mc = m[:, None]
            gi = jnp.minimum(
                jnp.minimum(jnp.where(vs[0] == mc, ws[0], BIGI),
                            jnp.where(vs[1] == mc, ws[1], BIGI)),
                jnp.minimum(jnp.where(vs[2] == mc, ws[2], BIGI),
                            jnp.where(vs[3] == mc, ws[3], BIGI)))
            gim = jnp.min(gi, axis=1)[:, None]
            for s in range(4):
                t5v[s] = jnp.where((vs[s] == mc) & (ws[s] == gim), jnp.inf,
                                   vs[s])
            resv[t] = mc
            resi[t] = gim
            return 0

        lax.fori_loop(0, KNB, _extract, 0)

        outd_ref[...] = jnp.concatenate([resv[t] for t in range(KNB)], axis=1)
        outi_ref[...] = jnp.concatenate([resi[t] for t in range(KNB)], axis=1)
        kth = resv[KNB - 1]
        vio = t5v[4] <= kth
        for c in range(NSTEPS):
            vio = vio | (jnp.sqrt(jnp.maximum(m3b[c], 1e-12)) <= kth)
        violq_ref[0, 0] = jnp.max(vio.astype(jnp.int32))


def _fast_call(q2, x, k2r, keys_p):
    last = NSTEPS - 1
    return pl.pallas_call(
        _fast_body,
        grid=(NSTEPS + 1,),
        in_specs=[
            pl.BlockSpec((Q, 1), lambda i: (0, 0)),
            pl.BlockSpec((Q, D), lambda i: (0, 0)),
            pl.BlockSpec((1, 1, BK), lambda i: (jnp.minimum(i, last), 0, 0)),
            pl.BlockSpec((BK, D), lambda i: (jnp.minimum(i, last), 0)),
        ],
        out_specs=[
            pl.BlockSpec((Q, KNB), lambda i: (0, 0)),
            pl.BlockSpec((Q, KNB), lambda i: (0, 0)),
            pl.BlockSpec(memory_space=pltpu.SMEM),
        ],
        out_shape=[
            jax.ShapeDtypeStruct((Q, KNB), jnp.float32),
            jax.ShapeDtypeStruct((Q, KNB), jnp.int32),
            jax.ShapeDtypeStruct((1, 1), jnp.int32),
        ],
        scratch_shapes=[
            pltpu.VMEM((2, Q, BK), jnp.float32),
            pltpu.VMEM((NSLOT, Q, 128), jnp.float32),
            pltpu.VMEM((NSLOT, Q, 128), jnp.int32),
            pltpu.VMEM((NSTEPS, Q, 128), jnp.float32),
            pltpu.VMEM((5, Q, 128), jnp.float32),
            pltpu.VMEM((5, Q, 128), jnp.int32),
            pltpu.VMEM((KNB, Q, 1), jnp.float32),
            pltpu.VMEM((KNB, Q, 1), jnp.int32),
        ],
    )(q2, x, k2r, keys_p)


def _exact_body(q2_ref, x_ref, k2_ref, ks_ref, outd_ref, outi_ref, bv, bi):
    i = pl.program_id(0)

    @pl.when(i == 0)
    def _init():
        bv[...] = jnp.full((Q, KNB), jnp.inf, jnp.float32)
        bi[...] = jnp.zeros((Q, KNB), jnp.int32)

    x = x_ref[...]
    ks = ks_ref[...]
    k2 = k2_ref[0]
    q2 = q2_ref[...]
    dot2 = lax.dot_general(x, ks, (((1,), (1,)), ((), ())),
                           preferred_element_type=jnp.float32)
    sq = (q2 + k2) + dot2
    d = jnp.sqrt(jnp.maximum(sq, 1e-12))

    W = KNB + BK
    cv = jnp.concatenate([bv[...], d], axis=1)
    iota_b = lax.broadcasted_iota(jnp.int32, (Q, BK), 1)
    ci = jnp.concatenate([bi[...], iota_b + i * BK], axis=1)
    iota_w = lax.broadcasted_iota(jnp.int32, (Q, W), 1)

    vals, idxs = [], []
    for _ in range(KNB):
        m = jnp.min(cv, axis=1)
        pos = jnp.min(jnp.where(cv == m[:, None], iota_w, W), axis=1)
        pm = iota_w == pos[:, None]
        gi = jnp.max(jnp.where(pm, ci, 0), axis=1)
        vals.append(m[:, None])
        idxs.append(gi[:, None])
        cv = jnp.where(pm, jnp.inf, cv)
    bv[...] = jnp.concatenate(vals, axis=1)
    bi[...] = jnp.concatenate(idxs, axis=1)

    @pl.when(i == NSTEPS - 1)
    def _fin():
        outd_ref[...] = bv[...]
        outi_ref[...] = bi[...]


def _exact_call(q2, x, k2r, keys_p):
    return pl.pallas_call(
        _exact_body,
        grid=(NSTEPS,),
        in_specs=[
            pl.BlockSpec((Q, 1), lambda i: (0, 0)),
            pl.BlockSpec((Q, D), lambda i: (0, 0)),
            pl.BlockSpec((1, 1, BK), lambda i: (i, 0, 0)),
            pl.BlockSpec((BK, D), lambda i: (i, 0)),
        ],
        out_specs=[
            pl.BlockSpec((Q, KNB), lambda i: (0, 0)),
            pl.BlockSpec((Q, KNB), lambda i: (0, 0)),
        ],
        out_shape=[
            jax.ShapeDtypeStruct((Q, KNB), jnp.float32),
            jax.ShapeDtypeStruct((Q, KNB), jnp.int32),
        ],
        scratch_shapes=[
            pltpu.VMEM((Q, KNB), jnp.float32),
            pltpu.VMEM((Q, KNB), jnp.int32),
        ],
    )(q2, x, k2r, keys_p)


def _gather_body(keys_hbm, cls_hbm, idx_hbm, emb_out, cls_out,
                 idx_v, rows_v, cls_v, sem_e, sem_c):
    wid = lax.axis_index("s") * NC + lax.axis_index("c")
    base = wid * BPW
    pltpu.sync_copy(idx_hbm.at[pl.ds(base, BPW)], idx_v)
    pltpu.async_copy(keys_hbm.at[idx_v], rows_v, sem_e).wait()
    pltpu.async_copy(cls_hbm.at[idx_v], cls_v, sem_c).wait()
    pltpu.sync_copy(rows_v, emb_out.at[pl.ds(base, BPW)])
    pltpu.sync_copy(cls_v, cls_out.at[pl.ds(base, BPW)])


@functools.lru_cache(maxsize=1)
def _sc_gather():
    return pl.kernel(
        _gather_body,
        out_type=[
            jax.ShapeDtypeStruct((Q * KNB, D), jnp.float32),
            jax.ShapeDtypeStruct((Q * KNB,), jnp.int32),
        ],
        mesh=plsc.VectorSubcoreMesh(core_axis_name="c", subcore_axis_name="s",
                                    num_cores=NC, num_subcores=NS),
        scratch_types=[
            pltpu.VMEM((BPW,), jnp.int32),
            pltpu.VMEM((BPW, D), jnp.float32),
            pltpu.VMEM((BPW,), jnp.int32),
            pltpu.SemaphoreType.DMA,
            pltpu.SemaphoreType.DMA,
        ],
    )


def kernel(x, keys, key_classes):
    q2 = jnp.sum(x * x, axis=1)
    k2 = jnp.sum(keys * keys, axis=1)
    q2c = q2[:, None]
    k2r = k2.reshape(NSTEPS, 1, BK)
    xm2 = -2.0 * x
    outd, outi, violq = _fast_call(q2c, xm2, k2r, keys)
    outd, outi = lax.cond(
        violq[0, 0] != 0,
        lambda: _exact_call(q2c, xm2, k2r, keys),
        lambda: (outd, outi),
    )
    emb, cls = _sc_gather()(keys, key_classes, outi.reshape(-1))
    return outd, emb.reshape(Q, KNB, D), cls.reshape(Q, KNB)
